# baseline (speedup 1.0000x reference)
import math

import jax
import jax.numpy as jnp
from jax import lax
from jax.experimental import pallas as pl
from jax.experimental.pallas import tpu as pltpu
from jax.experimental.pallas import tpu_sc as plsc

_SINK_SIZE = 16
_SLIDING_WINDOW = 4080
_NUM_KV_HEADS = 8
_HEAD_SIZE = 128
_BLOCK_SIZE = 16
_X = 8
_NUM_BLOCKS = 2048
_BATCH = 64
_ROPE_BASE = 10000.0

_CACHE_SIZE = float(_SLIDING_WINDOW + _SINK_SIZE)
_ROW = _NUM_KV_HEADS * (_HEAD_SIZE // _X) * _BLOCK_SIZE * _X
_NC = 2
_NS = 16
_NW = _NC * _NS
_ROWS_PER_W = _NUM_BLOCKS // _NW
_DEPTH = 4
_RB = 128
_NROWS2 = _NUM_BLOCKS * _RB


def _tables_body(btc_ref, btr_ref, posr_ref, cos_ref, sin_ref, enc_ref):
    btc = btc_ref[...]
    btr = btr_ref[...]
    posr = posr_ref[...]

    eq = btc == btr
    theta = jnp.maximum(posr.astype(jnp.float32) - _CACHE_SIZE, 0.0)
    angle = jnp.sum(
        jnp.where(eq, jnp.broadcast_to(theta, (_BATCH, _BATCH)), 0.0),
        axis=1, keepdims=True)

    lane = lax.broadcasted_iota(jnp.int32, (_BATCH, _HEAD_SIZE), 1)
    f = (lane // 16) * 8 + (lane % 16) % 8
    inv_freq = jnp.exp(
        f.astype(jnp.float32) * (-2.0 * math.log(_ROPE_BASE) / _HEAD_SIZE))
    ang = angle * inv_freq
    cos_ref[...] = jnp.cos(ang)
    sin_ref[...] = jnp.sin(ang)

    enc = jnp.where(angle > 0.0, btc, -1)
    enc_ref[...] = jnp.broadcast_to(enc, (_BATCH, 16))


def _make_tables(interpret=False):
    return pl.pallas_call(
        _tables_body,
        out_shape=(
            jax.ShapeDtypeStruct((_BATCH, _HEAD_SIZE), jnp.float32),
            jax.ShapeDtypeStruct((_BATCH, _HEAD_SIZE), jnp.float32),
            jax.ShapeDtypeStruct((_BATCH, 16), jnp.int32),
        ),
        interpret=interpret,
    )


def _sc_body(in_hbm, cos_hbm, sin_hbm, enc2_hbm, out_hbm,
             bufs, rot_v, cos_v, sin_v, enc2_v, in_sems, out_sems, rot_sem):
    cid = lax.axis_index("c")
    sid = lax.axis_index("s")
    wid = sid * _NC + cid
    base = wid * _ROWS_PER_W

    pltpu.sync_copy(enc2_hbm, enc2_v)

    def start_in(i):
        pltpu.make_async_copy(
            in_hbm.at[pl.ds((base + i) * _RB, _RB)],
            bufs.at[pl.ds((i % _DEPTH) * _RB, _RB)],
            in_sems.at[i % _DEPTH]).start()

    def wait_in(i):
        pltpu.make_async_copy(
            in_hbm.at[pl.ds((base + i) * _RB, _RB)],
            bufs.at[pl.ds((i % _DEPTH) * _RB, _RB)],
            in_sems.at[i % _DEPTH]).wait()

    def start_out(i):
        pltpu.make_async_copy(
            bufs.at[pl.ds((i % _DEPTH) * _RB, _RB)],
            out_hbm.at[pl.ds((base + i) * _RB, _RB)],
            out_sems.at[i % _DEPTH]).start()

    def wait_out(i):
        pltpu.make_async_copy(
            bufs.at[pl.ds((i % _DEPTH) * _RB, _RB)],
            out_hbm.at[pl.ds((base + i) * _RB, _RB)],
            out_sems.at[i % _DEPTH]).wait()

    for j in range(3):
        start_in(j)
    for i in range(_ROWS_PER_W):
        wait_in(i)
        start_out(i)
        if i >= 1:
            wait_out(i - 1)
        if i + 3 < _ROWS_PER_W:
            start_in(i + 3)
    wait_out(_ROWS_PER_W - 1)

    lanes = lax.broadcasted_iota(jnp.int32, (16,), 0)

    def p2(j, carry):
        r = base + j
        slot = jnp.int32(-1)
        for c in range(4):
            ch = enc2_v[c, :]
            slot = jnp.maximum(
                slot, jnp.max(jnp.where(ch == r, lanes + c * 16, -1)))

        @pl.when(slot >= 0)
        def _():
            pltpu.async_copy(
                in_hbm.at[pl.ds(r * _RB, _RB)], rot_v, rot_sem).wait()
            pltpu.sync_copy(cos_hbm.at[pl.ds(slot, 1)], cos_v)
            pltpu.sync_copy(sin_hbm.at[pl.ds(slot, 1)], sin_v)

            def body(hd, carry2):
                h = hd // _X
                dx = hd - h * _X
                g1 = h * 16 + dx
                g2 = g1 + 8
                cbase = dx * 16
                c = cos_v[0, pl.ds(cbase, 16)]
                s = sin_v[0, pl.ds(cbase, 16)]
                for v in range(8):
                    o = v * 16
                    k1 = rot_v[g1, pl.ds(o, 16)]
                    k2 = rot_v[g2, pl.ds(o, 16)]
                    rot_v[g1, pl.ds(o, 16)] = k1 * c - k2 * s
                    rot_v[g2, pl.ds(o, 16)] = k2 * c + k1 * s
                return carry2

            lax.fori_loop(0, _NUM_KV_HEADS * _X, body, 0)
            pltpu.async_copy(
                rot_v, out_hbm.at[pl.ds(r * _RB, _RB)], rot_sem).wait()

        return carry

    lax.fori_loop(0, _ROWS_PER_W, p2, 0)


def _make_sc_apply(interpret=False):
    mesh = plsc.VectorSubcoreMesh(
        core_axis_name="c", subcore_axis_name="s",
        num_cores=_NC, num_subcores=_NS)
    return pl.kernel(
        _sc_body,
        out_type=jax.ShapeDtypeStruct((_NROWS2, _RB), jnp.float32),
        mesh=mesh,
        compiler_params=pltpu.CompilerParams(
            needs_layout_passes=False, use_tc_tiling_on_sc=True),
        scratch_types=[
            pltpu.VMEM((_DEPTH * _RB, _RB), jnp.float32),
            pltpu.VMEM((_RB, _RB), jnp.float32),
            pltpu.VMEM((1, _HEAD_SIZE), jnp.float32),
            pltpu.VMEM((1, _HEAD_SIZE), jnp.float32),
            pltpu.VMEM((4, 16), jnp.int32),
            pltpu.SemaphoreType.DMA((_DEPTH,)),
            pltpu.SemaphoreType.DMA((_DEPTH,)),
            pltpu.SemaphoreType.DMA,
        ],
        interpret=interpret,
    )


def _kernel_impl(key_cache, block_tables, context_lens, positions,
                 interpret=False):
    del context_lens
    shape = key_cache.shape
    cache2 = key_cache.reshape(_NUM_BLOCKS, _ROW)
    btc = block_tables[:, :1]
    btr = btc.reshape(1, _BATCH)
    posr = positions.reshape(1, _BATCH)
    cos_t, sin_t, enc = _make_tables(interpret)(btc, btr, posr)
    enc2 = enc[:, 0].reshape(4, 16)
    cache3 = cache2.reshape(_NROWS2, _RB)
    out = _make_sc_apply(interpret)(cache3, cos_t, sin_t, enc2)
    return out.reshape(shape)


def kernel(key_cache, block_tables, context_lens, positions):
    return _kernel_impl(key_cache, block_tables, context_lens, positions)


# speedup vs baseline: 6.6232x; 6.6232x over previous
import math

import jax
import jax.numpy as jnp
from jax import lax
from jax.experimental import pallas as pl
from jax.experimental.pallas import tpu as pltpu
from jax.experimental.pallas import tpu_sc as plsc

_SINK_SIZE = 16
_SLIDING_WINDOW = 4080
_NUM_KV_HEADS = 8
_HEAD_SIZE = 128
_BLOCK_SIZE = 16
_X = 8
_NUM_BLOCKS = 2048
_BATCH = 64
_ROPE_BASE = 10000.0

_CACHE_SIZE = float(_SLIDING_WINDOW + _SINK_SIZE)
_HALF = _HEAD_SIZE // 2
_NROWS = 16384
_NC = 2
_NS = 16
_NW = _NC * _NS
_NGROUPS = _NUM_KV_HEADS * (_HEAD_SIZE // _X // 2)
_GPW = _NGROUPS // _NW
_CR = 4
_CPG = 128 // _CR
_CPW = _GPW * _CPG
_NSLOT = 3


def _tables_body(btc_ref, posc_ref, cos_ref, sin_ref):
    btc = btc_ref[...]
    posc = posc_ref[...]

    iota_b = lax.broadcasted_iota(jnp.int32, (_BATCH, _NUM_BLOCKS), 1)
    eq = btc == iota_b
    theta = jnp.maximum(posc.astype(jnp.float32) - _CACHE_SIZE, 0.0)
    masked = jnp.where(eq, jnp.broadcast_to(theta, (_BATCH, _NUM_BLOCKS)), 0.0)
    angle = jnp.sum(masked, axis=0, keepdims=True)

    fcol = lax.broadcasted_iota(jnp.int32, (_HALF, 1), 0).astype(jnp.float32)
    inv_freq = jnp.exp(fcol * (-2.0 * math.log(_ROPE_BASE) / _HEAD_SIZE))
    ang = inv_freq * angle
    cos_ref[...] = jnp.cos(ang)
    sin_ref[...] = jnp.sin(ang)


def _make_tables(interpret=False):
    return pl.pallas_call(
        _tables_body,
        out_shape=(
            jax.ShapeDtypeStruct((_HALF, _NUM_BLOCKS), jnp.float32),
            jax.ShapeDtypeStruct((_HALF, _NUM_BLOCKS), jnp.float32),
        ),
        interpret=interpret,
    )


def _sc_body(in_hbm, c_hbm, s_hbm, out_hbm,
             bufl, bufh, c_v, s_v, inl_sems, inh_sems, outl_sems, outh_sems):
    cid = lax.axis_index("c")
    sid = lax.axis_index("s")
    wid = sid * _NC + cid

    def rows_of(k):
        g = wid * _GPW + k // _CPG
        kc = k % _CPG
        h = g // 8
        dx = g - h * 8
        low = h * 2048 + dx * 128 + kc * _CR
        return low, dx, kc

    def start_in(k):
        low, _, _ = rows_of(k)
        slot = k % _NSLOT
        pltpu.make_async_copy(
            in_hbm.at[pl.ds(low, _CR)],
            bufl.at[pl.ds(slot * _CR, _CR)],
            inl_sems.at[slot]).start()
        pltpu.make_async_copy(
            in_hbm.at[pl.ds(low + 1024, _CR)],
            bufh.at[pl.ds(slot * _CR, _CR)],
            inh_sems.at[slot]).start()

    def wait_in(k):
        low, _, _ = rows_of(k)
        slot = k % _NSLOT
        pltpu.make_async_copy(
            in_hbm.at[pl.ds(low, _CR)],
            bufl.at[pl.ds(slot * _CR, _CR)],
            inl_sems.at[slot]).wait()
        pltpu.make_async_copy(
            in_hbm.at[pl.ds(low + 1024, _CR)],
            bufh.at[pl.ds(slot * _CR, _CR)],
            inh_sems.at[slot]).wait()

    def start_out(k):
        low, _, _ = rows_of(k)
        slot = k % _NSLOT
        pltpu.make_async_copy(
            bufl.at[pl.ds(slot * _CR, _CR)],
            out_hbm.at[pl.ds(low, _CR)],
            outl_sems.at[slot]).start()
        pltpu.make_async_copy(
            bufh.at[pl.ds(slot * _CR, _CR)],
            out_hbm.at[pl.ds(low + 1024, _CR)],
            outh_sems.at[slot]).start()

    def wait_out(k):
        low, _, _ = rows_of(k)
        slot = k % _NSLOT
        pltpu.make_async_copy(
            bufl.at[pl.ds(slot * _CR, _CR)],
            out_hbm.at[pl.ds(low, _CR)],
            outl_sems.at[slot]).wait()
        pltpu.make_async_copy(
            bufh.at[pl.ds(slot * _CR, _CR)],
            out_hbm.at[pl.ds(low + 1024, _CR)],
            outh_sems.at[slot]).wait()

    start_in(0)
    start_in(1)

    def step(k, carry):
        _, dx, kc = rows_of(k)
        slot = k % _NSLOT

        @pl.when(kc == 0)
        def _():
            pltpu.sync_copy(c_hbm.at[pl.ds(dx * 8, 8)], c_v)
            pltpu.sync_copy(s_hbm.at[pl.ds(dx * 8, 8)], s_v)

        wait_in(k)
        xb = (k % 2) * _CR

        def comp(v, carry2):
            o = v * 16
            for i in range(_CR):
                c = c_v[xb + i, pl.ds(o, 16)]
                s = s_v[xb + i, pl.ds(o, 16)]
                k1 = bufl[slot * _CR + i, pl.ds(o, 16)]
                k2 = bufh[slot * _CR + i, pl.ds(o, 16)]
                bufl[slot * _CR + i, pl.ds(o, 16)] = k1 * c - k2 * s
                bufh[slot * _CR + i, pl.ds(o, 16)] = k2 * c + k1 * s
            return carry2

        lax.fori_loop(0, _NUM_BLOCKS // 16, comp, 0)
        start_out(k)

        @pl.when(k >= 1)
        def _():
            wait_out(k - 1)

        @pl.when(k + 2 < _CPW)
        def _():
            start_in(k + 2)

        return carry

    lax.fori_loop(0, _CPW, step, 0)
    wait_out(_CPW - 1)


def _make_sc_apply(interpret=False):
    mesh = plsc.VectorSubcoreMesh(
        core_axis_name="c", subcore_axis_name="s",
        num_cores=_NC, num_subcores=_NS)
    return pl.kernel(
        _sc_body,
        out_type=jax.ShapeDtypeStruct((_NROWS, _NUM_BLOCKS), jnp.float32),
        mesh=mesh,
        compiler_params=pltpu.CompilerParams(
            needs_layout_passes=False, use_tc_tiling_on_sc=True),
        scratch_types=[
            pltpu.VMEM((_NSLOT * _CR, _NUM_BLOCKS), jnp.float32),
            pltpu.VMEM((_NSLOT * _CR, _NUM_BLOCKS), jnp.float32),
            pltpu.VMEM((8, _NUM_BLOCKS), jnp.float32),
            pltpu.VMEM((8, _NUM_BLOCKS), jnp.float32),
            pltpu.SemaphoreType.DMA((_NSLOT,)),
            pltpu.SemaphoreType.DMA((_NSLOT,)),
            pltpu.SemaphoreType.DMA((_NSLOT,)),
            pltpu.SemaphoreType.DMA((_NSLOT,)),
        ],
        interpret=interpret,
    )


def _kernel_impl(key_cache, block_tables, context_lens, positions,
                 interpret=False):
    del context_lens
    m = jnp.transpose(key_cache, (1, 2, 3, 4, 0)).reshape(_NROWS, _NUM_BLOCKS)
    btc = block_tables[:, :1]
    posc = positions.reshape(_BATCH, 1)
    cos_t, sin_t = _make_tables(interpret)(btc, posc)
    out = _make_sc_apply(interpret)(m, cos_t, sin_t)
    out5 = out.reshape(_NUM_KV_HEADS, _HEAD_SIZE // _X, _BLOCK_SIZE, _X,
                       _NUM_BLOCKS)
    return jnp.transpose(out5, (4, 0, 1, 2, 3))


def kernel(key_cache, block_tables, context_lens, positions):
    return _kernel_impl(key_cache, block_tables, context_lens, positions)
